# baseline (speedup 1.0000x reference)
import functools

import jax
import jax.numpy as jnp
from jax import lax
from jax.experimental import pallas as pl
from jax.experimental.pallas import tpu as pltpu
from jax.experimental.pallas import tpu_sc as plsc

B = 4096
L = 200
EMB = 16
HID = 64
NCLS = 100

NC = 2
NS = 16
NW = NC * NS
BPW = B // NW

R = 4
GROUP = R * L
NG = BPW // R
RCHUNKS = ((0, 128), (128, 72))


def _make_pool_kernel():
    mesh = plsc.VectorSubcoreMesh(core_axis_name="c", subcore_axis_name="s")

    @functools.partial(
        pl.kernel,
        mesh=mesh,
        out_type=jax.ShapeDtypeStruct((3, B, EMB), jnp.float32),
        compiler_params=pltpu.CompilerParams(use_tc_tiling_on_sc=False),
        scratch_types=[
            pltpu.VMEM((BPW, L), jnp.int32),
            pltpu.VMEM((GROUP, EMB), jnp.float32),
            pltpu.VMEM((GROUP, EMB), jnp.float32),
            pltpu.VMEM((BPW, EMB), jnp.float32),
            pltpu.SemaphoreType.DMA,
            pltpu.SemaphoreType.DMA,
        ],
    )
    def pool(shapes_hbm, colors_hbm, clusters_hbm, t0, t1, t2, out_hbm,
             idx_v, buf_a, buf_b, acc_v, sem_a, sem_b):
        wid = lax.axis_index("s") * NC + lax.axis_index("c")
        base = wid * BPW
        zero = jnp.zeros((EMB,), jnp.float32)

        for t, (idx_hbm, tab) in enumerate(
                ((shapes_hbm, t0), (colors_hbm, t1), (clusters_hbm, t2))):
            pltpu.sync_copy(idx_hbm.at[pl.ds(base, BPW)], idx_v)

            def issue(g, buf, sem):
                for r in range(R):
                    row = g * R + r
                    for off, n in RCHUNKS:
                        pltpu.async_copy(
                            tab.at[idx_v.at[row, pl.ds(off, n)]],
                            buf.at[pl.ds(r * L + off, n)], sem)

            def wait(buf, sem):
                pltpu.make_async_copy(
                    out_hbm.at[0].at[pl.ds(0, GROUP)], buf, sem).wait()

            def accum(g, buf):
                def row_body(r, carry):
                    def elem_body(j, accs):
                        a0, a1, a2, a3 = accs
                        b = r * L + j * 20
                        for u in range(0, 20, 4):
                            a0 = a0 + buf[b + u]
                            a1 = a1 + buf[b + u + 1]
                            a2 = a2 + buf[b + u + 2]
                            a3 = a3 + buf[b + u + 3]
                        return a0, a1, a2, a3
                    accs = lax.fori_loop(0, L // 20, elem_body,
                                         (zero, zero, zero, zero))
                    acc_v[g * R + r] = (accs[0] + accs[1]) + (accs[2] + accs[3])
                    return carry
                lax.fori_loop(0, R, row_body, 0)

            issue(0, buf_a, sem_a)
            issue(1, buf_b, sem_b)

            def pair_body(k, carry):
                wait(buf_a, sem_a)
                accum(2 * k, buf_a)
                issue(2 * k + 2, buf_a, sem_a)
                wait(buf_b, sem_b)
                accum(2 * k + 1, buf_b)
                issue(2 * k + 3, buf_b, sem_b)
                return carry

            lax.fori_loop(0, NG // 2 - 1, pair_body, 0)
            wait(buf_a, sem_a)
            accum(NG - 2, buf_a)
            wait(buf_b, sem_b)
            accum(NG - 1, buf_b)

            pltpu.sync_copy(acc_v, out_hbm.at[t].at[pl.ds(base, BPW)])

    return pool


_pool = _make_pool_kernel()


def _mlp_body(p0, p1, p2, m, w1, b1, w2, b2, o):
    s = jnp.dot(p0[0], w1[0], preferred_element_type=jnp.float32)
    s = s + jnp.dot(p1[0], w1[1], preferred_element_type=jnp.float32)
    s = s + jnp.dot(p2[0], w1[2], preferred_element_type=jnp.float32)
    msum = jnp.sum(m[...], axis=1, keepdims=True)
    h = jnp.maximum(s / msum + b1[...], 0.0)
    o[...] = jnp.dot(h, w2[...], preferred_element_type=jnp.float32) + b2[...]


def _mlp(psum3, mask, W1r, b1r, W2, b2r):
    BB = 512
    grid = (B // BB,)
    return pl.pallas_call(
        _mlp_body,
        grid=grid,
        in_specs=[
            pl.BlockSpec((1, BB, EMB), lambda b: (0, b, 0)),
            pl.BlockSpec((1, BB, EMB), lambda b: (1, b, 0)),
            pl.BlockSpec((1, BB, EMB), lambda b: (2, b, 0)),
            pl.BlockSpec((BB, L), lambda b: (b, 0)),
            pl.BlockSpec((3, EMB, HID), lambda b: (0, 0, 0)),
            pl.BlockSpec((1, HID), lambda b: (0, 0)),
            pl.BlockSpec((HID, NCLS), lambda b: (0, 0)),
            pl.BlockSpec((1, NCLS), lambda b: (0, 0)),
        ],
        out_specs=pl.BlockSpec((BB, NCLS), lambda b: (b, 0)),
        out_shape=jax.ShapeDtypeStruct((B, NCLS), jnp.float32),
    )(psum3, psum3, psum3, mask, W1r, b1r, W2, b2r)


def kernel(shapes, colors, clusters, mask, shape_table, color_table,
           cluster_table, W1, b1, W2, b2):
    psum3 = _pool(shapes, colors, clusters,
                  shape_table, color_table, cluster_table)
    return _mlp(psum3, mask, W1.reshape(3, EMB, HID),
                b1.reshape(1, HID), W2, b2.reshape(1, NCLS))


# speedup vs baseline: 1.0089x; 1.0089x over previous
import functools

import jax
import jax.numpy as jnp
from jax import lax
from jax.experimental import pallas as pl
from jax.experimental.pallas import tpu as pltpu
from jax.experimental.pallas import tpu_sc as plsc

B = 4096
L = 200
EMB = 16
HID = 64
NCLS = 100

NC = 2
NS = 16
NW = NC * NS
BPW = B // NW

P = 20
NCH = L // P
CROWS = P * BPW


def _make_pool_kernel():
    mesh = plsc.VectorSubcoreMesh(core_axis_name="c", subcore_axis_name="s")

    @functools.partial(
        pl.kernel,
        mesh=mesh,
        out_type=jax.ShapeDtypeStruct((3, B, EMB), jnp.float32),
        compiler_params=pltpu.CompilerParams(use_tc_tiling_on_sc=False),
        scratch_types=[
            pltpu.VMEM((L, BPW), jnp.int32),
            pltpu.VMEM((CROWS, EMB), jnp.float32),
            pltpu.VMEM((CROWS, EMB), jnp.float32),
            pltpu.VMEM((BPW, EMB), jnp.float32),
            pltpu.SemaphoreType.DMA,
            pltpu.SemaphoreType.DMA,
        ],
    )
    def pool(shapes_hbm, colors_hbm, clusters_hbm, t0, t1, t2, out_hbm,
             idx_v, buf_a, buf_b, acc_v, sem_a, sem_b):
        wid = lax.axis_index("s") * NC + lax.axis_index("c")
        base = wid * BPW
        zero = jnp.zeros((EMB,), jnp.float32)

        for t, (idx_hbm, tab) in enumerate(
                ((shapes_hbm, t0), (colors_hbm, t1), (clusters_hbm, t2))):
            pltpu.sync_copy(idx_hbm.at[:, pl.ds(base, BPW)], idx_v)

            def issue(c, buf, sem):
                for p in range(P):
                    pltpu.async_copy(
                        tab.at[idx_v.at[c * P + p, pl.ds(0, BPW)]],
                        buf.at[pl.ds(p * BPW, BPW)], sem)

            def wait(buf, sem):
                pltpu.make_async_copy(
                    out_hbm.at[0].at[pl.ds(0, CROWS)], buf, sem).wait()

            def accum(c, buf):
                def row_body(r, carry):
                    a0 = acc_v[r]
                    a1, a2, a3 = zero, zero, zero
                    for p in range(0, P, 4):
                        a0 = a0 + buf[p * BPW + r]
                        a1 = a1 + buf[(p + 1) * BPW + r]
                        a2 = a2 + buf[(p + 2) * BPW + r]
                        a3 = a3 + buf[(p + 3) * BPW + r]
                    acc_v[r] = (a0 + a1) + (a2 + a3)
                    return carry
                lax.fori_loop(0, BPW, row_body, 0)

            def zacc(r, carry):
                acc_v[r] = zero
                return carry
            lax.fori_loop(0, BPW, zacc, 0)

            issue(0, buf_a, sem_a)
            issue(1, buf_b, sem_b)

            def pair_body(k, carry):
                wait(buf_a, sem_a)
                accum(2 * k, buf_a)
                issue(2 * k + 2, buf_a, sem_a)
                wait(buf_b, sem_b)
                accum(2 * k + 1, buf_b)
                issue(2 * k + 3, buf_b, sem_b)
                return carry

            lax.fori_loop(0, NCH // 2 - 1, pair_body, 0)
            wait(buf_a, sem_a)
            accum(NCH - 2, buf_a)
            wait(buf_b, sem_b)
            accum(NCH - 1, buf_b)

            pltpu.sync_copy(acc_v, out_hbm.at[t].at[pl.ds(base, BPW)])

    return pool


_pool = _make_pool_kernel()


def _mlp_body(p0, p1, p2, m, w1, b1, w2, b2, o):
    s = jnp.dot(p0[0], w1[0], preferred_element_type=jnp.float32)
    s = s + jnp.dot(p1[0], w1[1], preferred_element_type=jnp.float32)
    s = s + jnp.dot(p2[0], w1[2], preferred_element_type=jnp.float32)
    msum = jnp.sum(m[...], axis=1, keepdims=True)
    h = jnp.maximum(s / msum + b1[...], 0.0)
    o[...] = jnp.dot(h, w2[...], preferred_element_type=jnp.float32) + b2[...]


def _mlp(psum3, mask, W1r, b1r, W2, b2r):
    BB = 512
    grid = (B // BB,)
    return pl.pallas_call(
        _mlp_body,
        grid=grid,
        in_specs=[
            pl.BlockSpec((1, BB, EMB), lambda b: (0, b, 0)),
            pl.BlockSpec((1, BB, EMB), lambda b: (1, b, 0)),
            pl.BlockSpec((1, BB, EMB), lambda b: (2, b, 0)),
            pl.BlockSpec((BB, L), lambda b: (b, 0)),
            pl.BlockSpec((3, EMB, HID), lambda b: (0, 0, 0)),
            pl.BlockSpec((1, HID), lambda b: (0, 0)),
            pl.BlockSpec((HID, NCLS), lambda b: (0, 0)),
            pl.BlockSpec((1, NCLS), lambda b: (0, 0)),
        ],
        out_specs=pl.BlockSpec((BB, NCLS), lambda b: (b, 0)),
        out_shape=jax.ShapeDtypeStruct((B, NCLS), jnp.float32),
    )(psum3, psum3, psum3, mask, W1r, b1r, W2, b2r)


def kernel(shapes, colors, clusters, mask, shape_table, color_table,
           cluster_table, W1, b1, W2, b2):
    psum3 = _pool(shapes.T, colors.T, clusters.T,
                  shape_table, color_table, cluster_table)
    return _mlp(psum3, mask, W1.reshape(3, EMB, HID),
                b1.reshape(1, HID), W2, b2.reshape(1, NCLS))
